# baseline (speedup 1.0000x reference)
import functools

import jax
import jax.numpy as jnp
from jax import lax
from jax.experimental import pallas as pl
from jax.experimental.pallas import tpu as pltpu
from jax.experimental.pallas import tpu_sc as plsc

D = 128
S = 16
NUM_WORKERS = 32
CHUNK = 8
NBUF = 2
R_SC = 8192
BR = 512
BP = 1024


def _sc_masked_sum(nev, msk):
    rows_per_w = R_SC // NUM_WORKERS
    n_chunks = rows_per_w // CHUNK
    assert n_chunks % NBUF == 0
    msk_flat = msk.reshape(-1)
    mesh = plsc.VectorSubcoreMesh(core_axis_name="c", subcore_axis_name="s")

    @functools.partial(
        pl.kernel,
        out_type=jax.ShapeDtypeStruct((R_SC, D), jnp.float32),
        mesh=mesh,
        scratch_types=[
            pltpu.VMEM((NBUF, CHUNK, S, D), jnp.float32),
            pltpu.VMEM((rows_per_w * S,), jnp.float32),
            pltpu.VMEM((NBUF, CHUNK, D), jnp.float32),
        ] + [pltpu.SemaphoreType.DMA] * (2 * NBUF),
        compiler_params=pltpu.CompilerParams(use_tc_tiling_on_sc=True),
    )
    def k(nev_hbm, msk_hbm, out_hbm, nev_v, msk_v, out_v, *sems):
        sem_in = sems[:NBUF]
        sem_out = sems[NBUF:]
        wid = lax.axis_index("s") * 2 + lax.axis_index("c")
        base = wid * rows_per_w
        pltpu.sync_copy(msk_hbm.at[pl.ds(base * S, rows_per_w * S)], msk_v)
        for b in range(NBUF):
            pltpu.async_copy(
                nev_hbm.at[pl.ds(base + b * CHUNK, CHUNK)], nev_v.at[b],
                sem_in[b])

        def outer(gg, carry):
            for b in range(NBUF):
                g = gg * NBUF + b
                row0 = base + g * CHUNK
                pltpu.make_async_copy(
                    nev_hbm.at[pl.ds(row0, CHUNK)], nev_v.at[b],
                    sem_in[b]).wait()
                @pl.when(g >= NBUF)
                def _():
                    pltpu.make_async_copy(
                        out_v.at[b], out_hbm.at[pl.ds(row0, CHUNK)],
                        sem_out[b]).wait()

                for r in range(CHUNK):
                    mrow = msk_v[pl.ds((g * CHUNK + r) * S, S)]
                    accs = [jnp.zeros((16,), jnp.float32)
                            for _ in range(D // 16)]
                    for s in range(S):
                        m = mrow[s]
                        for j in range(D // 16):
                            accs[j] = accs[j] + m * nev_v[b, r, s,
                                                          pl.ds(j * 16, 16)]
                    for j in range(D // 16):
                        out_v[b, r, pl.ds(j * 16, 16)] = accs[j]

                pltpu.async_copy(
                    out_v.at[b], out_hbm.at[pl.ds(row0, CHUNK)], sem_out[b])

                @pl.when(g + NBUF < n_chunks)
                def _():
                    pltpu.async_copy(
                        nev_hbm.at[pl.ds(row0 + NBUF * CHUNK, CHUNK)],
                        nev_v.at[b], sem_in[b])
            return carry

        lax.fori_loop(0, n_chunks // NBUF, outer, 0)
        for b in range(NBUF):
            pltpu.make_async_copy(
                out_v.at[b], out_hbm.at[pl.ds(base, CHUNK)], sem_out[b]).wait()

    return k(nev, msk_flat)


def _tc_masked_sum(nev2, msk):
    R = nev2.shape[0]
    r_tc = R - R_SC
    off = R_SC // BR

    def body(nev_ref, mk_ref, out_ref):
        acc = (nev_ref[:, :D] * mk_ref[:, 0:1]).astype(jnp.float32)
        for s in range(1, S):
            acc = acc + nev_ref[:, s * D:(s + 1) * D] * mk_ref[:, s:s + 1]
        out_ref[...] = acc

    return pl.pallas_call(
        body,
        grid=(r_tc // BR,),
        in_specs=[
            pl.BlockSpec((BR, S * D), lambda i: (off + i, 0)),
            pl.BlockSpec((BR, S), lambda i: (off + i, 0)),
        ],
        out_specs=pl.BlockSpec((BR, D), lambda i: (i, 0)),
        out_shape=jax.ShapeDtypeStruct((r_tc, D), jnp.float32),
    )(nev2, msk)


def _tc_linears(sv, ne, es_sc, es_tc, mk, w_ent, b_ent, w, b):
    N = sv.shape[0]
    n_sc_blocks = (R_SC // 2) // BP
    grid = (N // BP,)
    b_ent2 = b_ent.reshape(1, D)
    b2 = b.reshape(1, D)

    def body(sv_ref, ne_ref, essc_ref, estc_ref, mk_ref, wet_ref, web_ref,
             bent_ref, w1_ref, w2_ref, w3_ref, bb_ref, out_ref, nv_ref):
        i = pl.program_id(0)
        use_sc = i < n_sc_blocks
        es0 = jnp.where(use_sc, essc_ref[:, 0, :], estc_ref[:, 0, :])
        es1 = jnp.where(use_sc, essc_ref[:, 1, :], estc_ref[:, 1, :])
        wet = wet_ref[...]
        web = web_ref[...]
        bent = bent_ref[...]
        cnt0 = jnp.sum(mk_ref[:, 0, :], axis=1, keepdims=True)
        cnt1 = jnp.sum(mk_ref[:, 1, :], axis=1, keepdims=True)
        inv0 = 1.0 / jnp.where(cnt0 == 0.0, 1.0, cnt0)
        inv1 = 1.0 / jnp.where(cnt1 == 0.0, 1.0, cnt1)
        ea0 = es0 * inv0
        ea1 = es1 * inv1
        nv0 = (jnp.dot(ne_ref[:, 0, :], wet, preferred_element_type=jnp.float32)
               + jnp.dot(ea0, web, preferred_element_type=jnp.float32)
               + bent)
        nv1 = (jnp.dot(ne_ref[:, 1, :], wet, preferred_element_type=jnp.float32)
               + jnp.dot(ea1, web, preferred_element_type=jnp.float32)
               + bent)
        nv_ref[:, 0, :] = nv0
        nv_ref[:, 1, :] = nv1
        out_ref[...] = (
            jnp.dot(sv_ref[...], w1_ref[...], preferred_element_type=jnp.float32)
            + jnp.dot(nv0, w2_ref[...], preferred_element_type=jnp.float32)
            + jnp.dot(nv1, w3_ref[...], preferred_element_type=jnp.float32)
            + bb_ref[...])

    wspec = pl.BlockSpec((D, D), lambda i: (0, 0))
    bspec = pl.BlockSpec((1, D), lambda i: (0, 0))
    out, nv = pl.pallas_call(
        body,
        grid=grid,
        in_specs=[
            pl.BlockSpec((BP, D), lambda i: (i, 0)),
            pl.BlockSpec((BP, 2, D), lambda i: (i, 0, 0)),
            pl.BlockSpec((BP, 2, D),
                         lambda i: (jnp.minimum(i, n_sc_blocks - 1), 0, 0)),
            pl.BlockSpec((BP, 2, D),
                         lambda i: (jnp.maximum(i - n_sc_blocks, 0), 0, 0)),
            pl.BlockSpec((BP, 2, S), lambda i: (i, 0, 0)),
            wspec, wspec, bspec, wspec, wspec, wspec, bspec,
        ],
        out_specs=[
            pl.BlockSpec((BP, D), lambda i: (i, 0)),
            pl.BlockSpec((BP, 2, D), lambda i: (i, 0, 0)),
        ],
        out_shape=[
            jax.ShapeDtypeStruct((N, D), jnp.float32),
            jax.ShapeDtypeStruct((N, 2, D), jnp.float32),
        ],
    )(sv, ne, es_sc, es_tc, mk, w_ent[:D], w_ent[D:], b_ent2, w[:D],
      w[D:2 * D], w[2 * D:], b2)
    return out, nv


def kernel(self_vectors, neighbor_entity_vectors, neighbor_edge_vectors,
           masks, W_ent, b_ent, W, b):
    bs, p, d = self_vectors.shape
    n = bs * p
    nev = neighbor_edge_vectors.reshape(n * 2, S, d)
    nev2 = neighbor_edge_vectors.reshape(n * 2, S * d)
    msk = masks.reshape(n * 2, S)
    edge_sum_sc = _sc_masked_sum(nev, msk)
    edge_sum_tc = _tc_masked_sum(nev2, msk)
    sv = self_vectors.reshape(n, d)
    ne = neighbor_entity_vectors.reshape(n, 2, d)
    es_sc = edge_sum_sc.reshape(R_SC // 2, 2, d)
    es_tc = edge_sum_tc.reshape((n * 2 - R_SC) // 2, 2, d)
    mk = masks.reshape(n, 2, S)
    out, nv = _tc_linears(sv, ne, es_sc, es_tc, mk, W_ent, b_ent, W, b)
    return (out.reshape(bs, p, d), nv.reshape(bs, p, 2, d))


# speedup vs baseline: 1.5943x; 1.5943x over previous
import functools

import jax
import jax.numpy as jnp
from jax import lax
from jax.experimental import pallas as pl
from jax.experimental.pallas import tpu as pltpu
from jax.experimental.pallas import tpu_sc as plsc

D = 128
S = 16
NUM_WORKERS = 32
CHUNK = 8
NBUF = 2
R_SC = 8192
BR = 512
BP = 1024


def _sc_masked_sum(nev, msk):
    rows_per_w = R_SC // NUM_WORKERS
    n_chunks = rows_per_w // CHUNK
    assert n_chunks % NBUF == 0
    msk_flat = msk.reshape(-1)
    mesh = plsc.VectorSubcoreMesh(core_axis_name="c", subcore_axis_name="s")

    @functools.partial(
        pl.kernel,
        out_type=jax.ShapeDtypeStruct((R_SC, D), jnp.float32),
        mesh=mesh,
        scratch_types=[
            pltpu.VMEM((NBUF, CHUNK, S, D), jnp.float32),
            pltpu.VMEM((rows_per_w * S,), jnp.float32),
            pltpu.VMEM((NBUF, CHUNK, D), jnp.float32),
        ] + [pltpu.SemaphoreType.DMA] * (2 * NBUF),
        compiler_params=pltpu.CompilerParams(use_tc_tiling_on_sc=True),
    )
    def k(nev_hbm, msk_hbm, out_hbm, nev_v, msk_v, out_v, *sems):
        sem_in = sems[:NBUF]
        sem_out = sems[NBUF:]
        wid = lax.axis_index("s") * 2 + lax.axis_index("c")
        base = wid * rows_per_w
        pltpu.sync_copy(msk_hbm.at[pl.ds(base * S, rows_per_w * S)], msk_v)
        for b in range(NBUF):
            pltpu.async_copy(
                nev_hbm.at[pl.ds(base + b * CHUNK, CHUNK)], nev_v.at[b],
                sem_in[b])

        def outer(gg, carry):
            for b in range(NBUF):
                g = gg * NBUF + b
                row0 = base + g * CHUNK
                pltpu.make_async_copy(
                    nev_hbm.at[pl.ds(row0, CHUNK)], nev_v.at[b],
                    sem_in[b]).wait()
                @pl.when(g >= NBUF)
                def _():
                    pltpu.make_async_copy(
                        out_v.at[b], out_hbm.at[pl.ds(row0, CHUNK)],
                        sem_out[b]).wait()

                for r in range(CHUNK):
                    mrow = msk_v[pl.ds((g * CHUNK + r) * S, S)]
                    accs = [jnp.zeros((16,), jnp.float32)
                            for _ in range(D // 16)]
                    for s in range(S):
                        m = mrow[s]
                        for j in range(D // 16):
                            accs[j] = accs[j] + m * nev_v[b, r, s,
                                                          pl.ds(j * 16, 16)]
                    for j in range(D // 16):
                        out_v[b, r, pl.ds(j * 16, 16)] = accs[j]

                pltpu.async_copy(
                    out_v.at[b], out_hbm.at[pl.ds(row0, CHUNK)], sem_out[b])

                @pl.when(g + NBUF < n_chunks)
                def _():
                    pltpu.async_copy(
                        nev_hbm.at[pl.ds(row0 + NBUF * CHUNK, CHUNK)],
                        nev_v.at[b], sem_in[b])
            return carry

        lax.fori_loop(0, n_chunks // NBUF, outer, 0)
        for b in range(NBUF):
            pltpu.make_async_copy(
                out_v.at[b], out_hbm.at[pl.ds(base, CHUNK)], sem_out[b]).wait()

    return k(nev, msk_flat)


def _tc_masked_sum(nev, msk):
    R = nev.shape[0]
    r_tc = R - R_SC
    off = R_SC // BR

    def body(nev_ref, mk_ref, out_ref):
        m = mk_ref[...][:, :, None]
        out_ref[...] = jnp.sum(nev_ref[...] * m, axis=1)

    return pl.pallas_call(
        body,
        grid=(r_tc // BR,),
        in_specs=[
            pl.BlockSpec((BR, S, D), lambda i: (off + i, 0, 0)),
            pl.BlockSpec((BR, S), lambda i: (off + i, 0)),
        ],
        out_specs=pl.BlockSpec((BR, D), lambda i: (i, 0)),
        out_shape=jax.ShapeDtypeStruct((r_tc, D), jnp.float32),
    )(nev, msk)


def _tc_linears(sv, ne, es_sc, es_tc, mk, w_ent, b_ent, w, b):
    N = sv.shape[0]
    n_sc_blocks = (R_SC // 2) // BP
    grid = (N // BP,)
    b_ent2 = b_ent.reshape(1, D)
    b2 = b.reshape(1, D)

    def body(sv_ref, ne_ref, essc_ref, estc_ref, mk_ref, wet_ref, web_ref,
             bent_ref, w1_ref, w2_ref, w3_ref, bb_ref, out_ref, nv_ref):
        i = pl.program_id(0)
        use_sc = i < n_sc_blocks
        es0 = jnp.where(use_sc, essc_ref[:, 0, :], estc_ref[:, 0, :])
        es1 = jnp.where(use_sc, essc_ref[:, 1, :], estc_ref[:, 1, :])
        wet = wet_ref[...]
        web = web_ref[...]
        bent = bent_ref[...]
        cnt0 = jnp.sum(mk_ref[:, 0, :], axis=1, keepdims=True)
        cnt1 = jnp.sum(mk_ref[:, 1, :], axis=1, keepdims=True)
        inv0 = 1.0 / jnp.where(cnt0 == 0.0, 1.0, cnt0)
        inv1 = 1.0 / jnp.where(cnt1 == 0.0, 1.0, cnt1)
        ea0 = es0 * inv0
        ea1 = es1 * inv1
        nv0 = (jnp.dot(ne_ref[:, 0, :], wet, preferred_element_type=jnp.float32)
               + jnp.dot(ea0, web, preferred_element_type=jnp.float32)
               + bent)
        nv1 = (jnp.dot(ne_ref[:, 1, :], wet, preferred_element_type=jnp.float32)
               + jnp.dot(ea1, web, preferred_element_type=jnp.float32)
               + bent)
        nv_ref[:, 0, :] = nv0
        nv_ref[:, 1, :] = nv1
        out_ref[...] = (
            jnp.dot(sv_ref[...], w1_ref[...], preferred_element_type=jnp.float32)
            + jnp.dot(nv0, w2_ref[...], preferred_element_type=jnp.float32)
            + jnp.dot(nv1, w3_ref[...], preferred_element_type=jnp.float32)
            + bb_ref[...])

    wspec = pl.BlockSpec((D, D), lambda i: (0, 0))
    bspec = pl.BlockSpec((1, D), lambda i: (0, 0))
    out, nv = pl.pallas_call(
        body,
        grid=grid,
        in_specs=[
            pl.BlockSpec((BP, D), lambda i: (i, 0)),
            pl.BlockSpec((BP, 2, D), lambda i: (i, 0, 0)),
            pl.BlockSpec((BP, 2, D),
                         lambda i: (jnp.minimum(i, n_sc_blocks - 1), 0, 0)),
            pl.BlockSpec((BP, 2, D),
                         lambda i: (jnp.maximum(i - n_sc_blocks, 0), 0, 0)),
            pl.BlockSpec((BP, 2, S), lambda i: (i, 0, 0)),
            wspec, wspec, bspec, wspec, wspec, wspec, bspec,
        ],
        out_specs=[
            pl.BlockSpec((BP, D), lambda i: (i, 0)),
            pl.BlockSpec((BP, 2, D), lambda i: (i, 0, 0)),
        ],
        out_shape=[
            jax.ShapeDtypeStruct((N, D), jnp.float32),
            jax.ShapeDtypeStruct((N, 2, D), jnp.float32),
        ],
    )(sv, ne, es_sc, es_tc, mk, w_ent[:D], w_ent[D:], b_ent2, w[:D],
      w[D:2 * D], w[2 * D:], b2)
    return out, nv


def kernel(self_vectors, neighbor_entity_vectors, neighbor_edge_vectors,
           masks, W_ent, b_ent, W, b):
    bs, p, d = self_vectors.shape
    n = bs * p
    nev = neighbor_edge_vectors.reshape(n * 2, S, d)
    msk = masks.reshape(n * 2, S)
    edge_sum_sc = _sc_masked_sum(nev, msk)
    edge_sum_tc = _tc_masked_sum(nev, msk)
    sv = self_vectors.reshape(n, d)
    ne = neighbor_entity_vectors.reshape(n, 2, d)
    es_sc = edge_sum_sc.reshape(R_SC // 2, 2, d)
    es_tc = edge_sum_tc.reshape((n * 2 - R_SC) // 2, 2, d)
    mk = masks.reshape(n, 2, S)
    out, nv = _tc_linears(sv, ne, es_sc, es_tc, mk, W_ent, b_ent, W, b)
    return (out.reshape(bs, p, d), nv.reshape(bs, p, 2, d))
